# baseline (speedup 1.0000x reference)
import jax
import jax.numpy as jnp
from jax.experimental import pallas as pl
from jax.experimental.pallas import tpu as pltpu


def _dma_copy_kernel(x_hbm, o_hbm, sem):
    cp = pltpu.make_async_copy(x_hbm, o_hbm, sem)
    cp.start()
    cp.wait()


def kernel(method_soft, label, features):
    del label, features
    n, d = method_soft.shape
    x = method_soft.reshape(n * d)
    y = pl.pallas_call(
        _dma_copy_kernel,
        out_shape=jax.ShapeDtypeStruct((n * d,), method_soft.dtype),
        in_specs=[pl.BlockSpec(memory_space=pl.ANY)],
        out_specs=pl.BlockSpec(memory_space=pl.ANY),
        scratch_shapes=[pltpu.SemaphoreType.DMA],
    )(x)
    return y.reshape(n, d)


# speedup vs baseline: 13.3919x; 13.3919x over previous
import jax
import jax.numpy as jnp
from jax.experimental import pallas as pl

_GRID = 8


def _copy_kernel(x_ref, o_ref):
    o_ref[...] = x_ref[...]


def kernel(method_soft, label, features):
    del label, features
    n, d = method_soft.shape
    xt = method_soft.T
    block = (d, n // _GRID)
    yt = pl.pallas_call(
        _copy_kernel,
        out_shape=jax.ShapeDtypeStruct((d, n), method_soft.dtype),
        grid=(_GRID,),
        in_specs=[pl.BlockSpec(block, lambda i: (0, i))],
        out_specs=pl.BlockSpec(block, lambda i: (0, i)),
    )(xt)
    return yt.T


# speedup vs baseline: 31.1352x; 2.3249x over previous
import jax
import jax.numpy as jnp
from jax.experimental import pallas as pl
from jax.experimental.pallas import tpu as pltpu

_C = 8


def _copy_kernel(x_hbm, o_hbm, *rest):
    bufs, sems_in, sems_out = rest[:_C], rest[_C:2 * _C], rest[2 * _C:]
    w = x_hbm.shape[1] // _C
    cps_in = [
        pltpu.make_async_copy(x_hbm.at[:, pl.ds(k * w, w)], bufs[k], sems_in[k])
        for k in range(_C)
    ]
    cps_out = [
        pltpu.make_async_copy(bufs[k], o_hbm.at[:, pl.ds(k * w, w)], sems_out[k])
        for k in range(_C)
    ]
    for cp in cps_in:
        cp.start()
    for k in range(_C):
        cps_in[k].wait()
        cps_out[k].start()
    for cp in cps_out:
        cp.wait()


def kernel(method_soft, label, features):
    del label, features
    n, d = method_soft.shape
    xt = method_soft.T
    yt = pl.pallas_call(
        _copy_kernel,
        out_shape=jax.ShapeDtypeStruct((d, n), method_soft.dtype),
        in_specs=[pl.BlockSpec(memory_space=pl.ANY)],
        out_specs=pl.BlockSpec(memory_space=pl.ANY),
        scratch_shapes=(
            [pltpu.VMEM((d, n // _C), method_soft.dtype)] * _C
            + [pltpu.SemaphoreType.DMA] * (2 * _C)
        ),
    )(xt)
    return yt.T
